# baseline (speedup 1.0000x reference)
import jax
import jax.numpy as jnp
from jax.experimental import pallas as pl

B, T, N, FIN, FOUT = 4, 8, 512, 128, 64
BT = B * T


def _gat_body(inp_ref, adjt_ref, w_ref, a1t_ref, a2t_ref, out_ref):
    x = inp_ref[0]
    h = jnp.dot(x, w_ref[...], preferred_element_type=jnp.float32)
    mask_t = (adjt_ref[...] > 0).astype(jnp.float32)
    h2 = jnp.dot(mask_t, h, preferred_element_type=jnp.float32)
    score = (jnp.sum(h * a1t_ref[...], axis=1, keepdims=True)
             + jnp.sum(h2 * a2t_ref[...], axis=1, keepdims=True))
    colsum = jnp.sum(h, axis=0, keepdims=True)
    out_ref[0] = jnp.maximum(score * colsum, 0.0)


def kernel(inp, adj, W, a):
    f = W.shape[1]
    inp_r = inp.reshape(BT, N, FIN)
    adj_t = adj.T
    a_t = a.T
    a1t = a_t[:, :f]
    a2t = a_t[:, f:]

    out = pl.pallas_call(
        _gat_body,
        grid=(BT,),
        in_specs=[
            pl.BlockSpec((1, N, FIN), lambda i: (i, 0, 0)),
            pl.BlockSpec((N, N), lambda i: (0, 0)),
            pl.BlockSpec((FIN, f), lambda i: (0, 0)),
            pl.BlockSpec((N, f), lambda i: (0, 0)),
            pl.BlockSpec((N, f), lambda i: (0, 0)),
        ],
        out_specs=pl.BlockSpec((1, N, f), lambda i: (i, 0, 0)),
        out_shape=jax.ShapeDtypeStruct((BT, N, f), jnp.float32),
    )(inp_r, adj_t, W, a1t, a2t)

    return out.reshape(B, T, N, f)


# speedup vs baseline: 1.0279x; 1.0279x over previous
import jax
import jax.numpy as jnp
from jax.experimental import pallas as pl

B, T, N, FIN, FOUT = 4, 8, 512, 128, 64
BT = B * T


def _gat_body(inp_ref, mask_ref, w_ref, a1_ref, a2_ref, out_ref):
    xb = inp_ref[0].astype(jnp.bfloat16)
    wb = w_ref[...].astype(jnp.bfloat16)
    ht = jax.lax.dot_general(wb, xb, (((0,), (1,)), ((), ())),
                             preferred_element_type=jnp.float32)
    h2t = jnp.dot(ht.astype(jnp.bfloat16), mask_ref[...],
                  preferred_element_type=jnp.float32)
    score = (jnp.sum(ht * a1_ref[...], axis=0)
             + jnp.sum(h2t * a2_ref[...], axis=0))
    colsum = jnp.sum(ht, axis=1)
    out_ref[0] = jnp.maximum(score[:, None] * colsum[None, :], 0.0)


def kernel(inp, adj, W, a):
    f = W.shape[1]
    inp_r = inp.reshape(BT, N, FIN)
    mask_b = (adj > 0).astype(jnp.bfloat16)
    a1 = a[:f, :]
    a2 = a[f:, :]

    out = pl.pallas_call(
        _gat_body,
        grid=(BT,),
        in_specs=[
            pl.BlockSpec((1, N, FIN), lambda i: (i, 0, 0)),
            pl.BlockSpec((N, N), lambda i: (0, 0)),
            pl.BlockSpec((FIN, f), lambda i: (0, 0)),
            pl.BlockSpec((f, N), lambda i: (0, 0)),
            pl.BlockSpec((f, N), lambda i: (0, 0)),
        ],
        out_specs=pl.BlockSpec((1, N, f), lambda i: (i, 0, 0)),
        out_shape=jax.ShapeDtypeStruct((BT, N, f), jnp.float32),
    )(inp_r, mask_b, W, a1, a2)

    return out.reshape(B, T, N, f)


# speedup vs baseline: 1.5453x; 1.5033x over previous
import jax
import jax.numpy as jnp
from jax.experimental import pallas as pl

B, T, N, FIN, FOUT = 4, 8, 512, 128, 64
BT = B * T


BT_PER = 4


def _gat_body(inp_ref, mask_ref, w_ref, a1_ref, a2_ref, out_ref):
    wb = w_ref[...].astype(jnp.bfloat16)
    mask = mask_ref[...]
    a1 = a1_ref[...]
    a2 = a2_ref[...]
    for k in range(BT_PER):
        xb = inp_ref[k].astype(jnp.bfloat16)
        ht = jax.lax.dot_general(wb, xb, (((0,), (1,)), ((), ())),
                                 preferred_element_type=jnp.float32)
        h2t = jnp.dot(ht.astype(jnp.bfloat16), mask,
                      preferred_element_type=jnp.float32)
        score = (jnp.sum(ht * a1, axis=0)
                 + jnp.sum(h2t * a2, axis=0))
        colsum = jnp.sum(ht, axis=1)
        out_ref[k] = jnp.maximum(score[:, None] * colsum[None, :], 0.0)


def kernel(inp, adj, W, a):
    f = W.shape[1]
    inp_r = inp.reshape(BT, N, FIN)
    mask_b = (adj > 0).astype(jnp.bfloat16)
    a1 = a[:f, :]
    a2 = a[f:, :]

    out = pl.pallas_call(
        _gat_body,
        grid=(BT // BT_PER,),
        in_specs=[
            pl.BlockSpec((BT_PER, N, FIN), lambda i: (i, 0, 0)),
            pl.BlockSpec((N, N), lambda i: (0, 0)),
            pl.BlockSpec((FIN, f), lambda i: (0, 0)),
            pl.BlockSpec((f, N), lambda i: (0, 0)),
            pl.BlockSpec((f, N), lambda i: (0, 0)),
        ],
        out_specs=pl.BlockSpec((BT_PER, N, f), lambda i: (i, 0, 0)),
        out_shape=jax.ShapeDtypeStruct((BT, N, f), jnp.float32),
    )(inp_r, mask_b, W, a1, a2)

    return out.reshape(B, T, N, f)


# speedup vs baseline: 1.6610x; 1.0749x over previous
import jax
import jax.numpy as jnp
from jax.experimental import pallas as pl

B, T, N, FIN, FOUT = 4, 8, 512, 128, 64
BT = B * T


BT_PER = 8


def _gat_body(inp_ref, mask_ref, w_ref, a1_ref, a2_ref, out_ref):
    wb = w_ref[...].astype(jnp.bfloat16)
    mask = mask_ref[...]
    a1 = a1_ref[...]
    a2 = a2_ref[...]
    for k in range(BT_PER):
        xb = inp_ref[k].astype(jnp.bfloat16)
        ht = jax.lax.dot_general(wb, xb, (((0,), (1,)), ((), ())),
                                 preferred_element_type=jnp.float32)
        h2t = jnp.dot(ht.astype(jnp.bfloat16), mask,
                      preferred_element_type=jnp.float32)
        score = (jnp.sum(ht * a1, axis=0)
                 + jnp.sum(h2t * a2, axis=0))
        colsum = jnp.sum(ht, axis=1)
        out_ref[k] = jnp.maximum(score[:, None] * colsum[None, :], 0.0)


def kernel(inp, adj, W, a):
    f = W.shape[1]
    inp_r = inp.reshape(BT, N, FIN)
    mask_b = (adj > 0).astype(jnp.bfloat16)
    a1 = a[:f, :]
    a2 = a[f:, :]

    out = pl.pallas_call(
        _gat_body,
        grid=(BT // BT_PER,),
        in_specs=[
            pl.BlockSpec((BT_PER, N, FIN), lambda i: (i, 0, 0)),
            pl.BlockSpec((N, N), lambda i: (0, 0)),
            pl.BlockSpec((FIN, f), lambda i: (0, 0)),
            pl.BlockSpec((f, N), lambda i: (0, 0)),
            pl.BlockSpec((f, N), lambda i: (0, 0)),
        ],
        out_specs=pl.BlockSpec((BT_PER, N, f), lambda i: (i, 0, 0)),
        out_shape=jax.ShapeDtypeStruct((BT, N, f), jnp.float32),
    )(inp_r, mask_b, W, a1, a2)

    return out.reshape(B, T, N, f)
